# baseline (speedup 1.0000x reference)
import functools

import jax
import jax.numpy as jnp
from jax import lax
from jax.experimental import pallas as pl
from jax.experimental.pallas import tpu as pltpu
from jax.experimental.pallas import tpu_sc as plsc

N, D, Q = 16384, 64, 512
NBLK = 2048
INV = 1.0 / 16384.0

SC_NC, SC_NS = 2, 16
SC_NW = SC_NC * SC_NS
QPW = Q // SC_NW
L = 16


def _match_argmin_body(x_ref, xs_ref, out_ref):
    blk = pl.program_id(0)
    xq = x_ref[...]
    xb = xs_ref[...]
    qb2 = (xq + xq).astype(jnp.bfloat16)
    db = xb.astype(jnp.bfloat16)
    g2 = lax.dot_general(qb2, db, (((1,), (1,)), ((), ())),
                         preferred_element_type=jnp.float32)
    qn = jnp.sum(xq * xq, axis=1, keepdims=True)
    sq = db * db
    ones = jnp.ones((1, D), jnp.bfloat16)
    xn = lax.dot_general(ones, sq, (((1,), (1,)), ((), ())),
                         preferred_element_type=jnp.float32)
    iotaf = lax.broadcasted_iota(jnp.int32, (1, NBLK), 1).astype(
        jnp.float32) * INV + blk.astype(jnp.float32) * (NBLK * INV)
    key = ((qn + xn) - g2) + iotaf
    m = jnp.min(key, axis=1, keepdims=True)

    @pl.when(blk == 0)
    def _():
        out_ref[...] = m

    @pl.when(blk > 0)
    def _():
        out_ref[...] = jnp.minimum(out_ref[...], m)


def _tc_match_argmin(X, xs, interpret=False):
    grid = (N // NBLK,)
    return pl.pallas_call(
        _match_argmin_body,
        grid=grid,
        in_specs=[
            pl.BlockSpec((Q, D), lambda i: (0, 0)),
            pl.BlockSpec((NBLK, D), lambda i: (i, 0)),
        ],
        out_specs=pl.BlockSpec((Q, 1), lambda i: (0, 0)),
        out_shape=jax.ShapeDtypeStruct((Q, 1), jnp.float32),
        interpret=interpret,
    )(X, xs)


def _sc_gather(ys_flat, mkey):
    mesh = plsc.VectorSubcoreMesh(core_axis_name="c", subcore_axis_name="s")

    @functools.partial(
        pl.kernel,
        mesh=mesh,
        compiler_params=pltpu.CompilerParams(use_tc_tiling_on_sc=False),
        out_type=jax.ShapeDtypeStruct((3, Q), jnp.float32),
        scratch_types=[
            pltpu.VMEM((QPW,), jnp.float32),
            pltpu.VMEM((3, QPW), jnp.int32),
            pltpu.VMEM((3, QPW), jnp.float32),
            pltpu.SemaphoreType.DMA,
        ],
    )
    def k(ys_hbm, mkey_hbm, out_hbm, mk_v, idx3_v, rows_v, sem):
        wid = lax.axis_index("s") * SC_NC + lax.axis_index("c")
        base = wid * QPW
        pltpu.sync_copy(mkey_hbm.at[pl.ds(base, QPW)], mk_v)
        v = jnp.clip((mk_v[...] * 16384.0).astype(jnp.int32), 0, N - 1)
        v3 = v * 3
        for c in range(3):
            idx3_v[c, :] = v3 + c
            pltpu.async_copy(ys_hbm.at[idx3_v.at[c]], rows_v.at[c], sem)
        pltpu.make_async_copy(ys_hbm.at[idx3_v.at[0]], rows_v.at[0], sem).wait()
        pltpu.make_async_copy(ys_hbm.at[idx3_v.at[1]], rows_v.at[1], sem).wait()
        pltpu.make_async_copy(ys_hbm.at[idx3_v.at[2]], rows_v.at[2], sem).wait()
        pltpu.sync_copy(rows_v, out_hbm.at[:, pl.ds(base, QPW)])

    return k(ys_flat, mkey)


def kernel(X, xs, ys):
    m = _tc_match_argmin(X, xs)
    out3q = _sc_gather(ys.reshape(-1), m.reshape(-1))
    return out3q.T


# speedup vs baseline: 1.5427x; 1.5427x over previous
import functools

import jax
import jax.numpy as jnp
from jax import lax
from jax.experimental import pallas as pl
from jax.experimental.pallas import tpu as pltpu
from jax.experimental.pallas import tpu_sc as plsc

N, D, Q = 16384, 64, 512
NBLK = 2048
INV = 1.0 / 16384.0

SC_NC, SC_NS = 2, 16
SC_NW = SC_NC * SC_NS
QPW = Q // SC_NW
L = 16


def _match_argmin_body(xt_ref, xst_ref, out_ref):
    blk = pl.program_id(0)
    xqt = xt_ref[...]
    xbt = xst_ref[...]
    qb2t = (xqt + xqt).astype(jnp.bfloat16)
    dbt = xbt.astype(jnp.bfloat16)
    g2 = lax.dot_general(qb2t, dbt, (((0,), (0,)), ((), ())),
                         preferred_element_type=jnp.float32)
    sqq = xqt * xqt
    ones_col = jnp.ones((D, 1), jnp.float32)
    qn = lax.dot_general(sqq, ones_col, (((0,), (0,)), ((), ())),
                         preferred_element_type=jnp.float32)
    sq = dbt * dbt
    ones_row = jnp.ones((1, D), jnp.bfloat16)
    xn = lax.dot_general(ones_row, sq, (((1,), (0,)), ((), ())),
                         preferred_element_type=jnp.float32)
    iotaf = lax.broadcasted_iota(jnp.int32, (1, NBLK), 1).astype(
        jnp.float32) * INV + blk.astype(jnp.float32) * (NBLK * INV)
    key = ((qn + xn) - g2) + iotaf
    m = jnp.min(key, axis=1)

    @pl.when(blk == 0)
    def _():
        out_ref[...] = m

    @pl.when(blk > 0)
    def _():
        out_ref[...] = jnp.minimum(out_ref[...], m)


def _tc_match_argmin(Xt, xst, interpret=False):
    grid = (N // NBLK,)
    return pl.pallas_call(
        _match_argmin_body,
        grid=grid,
        in_specs=[
            pl.BlockSpec((D, Q), lambda i: (0, 0)),
            pl.BlockSpec((D, NBLK), lambda i: (0, i)),
        ],
        out_specs=pl.BlockSpec((Q,), lambda i: (0,)),
        out_shape=jax.ShapeDtypeStruct((Q,), jnp.float32),
        interpret=interpret,
    )(Xt, xst)


def _sc_gather(yst_flat, mkey):
    mesh = plsc.VectorSubcoreMesh(core_axis_name="c", subcore_axis_name="s")

    @functools.partial(
        pl.kernel,
        mesh=mesh,
        compiler_params=pltpu.CompilerParams(use_tc_tiling_on_sc=False),
        out_type=jax.ShapeDtypeStruct((3, Q), jnp.float32),
        scratch_types=[
            pltpu.VMEM((QPW,), jnp.float32),
            pltpu.VMEM((3, QPW), jnp.int32),
            pltpu.VMEM((3, QPW), jnp.float32),
            pltpu.SemaphoreType.DMA,
        ],
    )
    def k(ys_hbm, mkey_hbm, out_hbm, mk_v, idx3_v, rows_v, sem):
        wid = lax.axis_index("s") * SC_NC + lax.axis_index("c")
        base = wid * QPW
        pltpu.sync_copy(mkey_hbm.at[pl.ds(base, QPW)], mk_v)
        v = jnp.clip((mk_v[...] * 16384.0).astype(jnp.int32), 0, N - 1)
        for c in range(3):
            idx3_v[c, :] = v + c * N
            pltpu.async_copy(ys_hbm.at[idx3_v.at[c]], rows_v.at[c], sem)
        pltpu.make_async_copy(ys_hbm.at[idx3_v.at[0]], rows_v.at[0], sem).wait()
        pltpu.make_async_copy(ys_hbm.at[idx3_v.at[1]], rows_v.at[1], sem).wait()
        pltpu.make_async_copy(ys_hbm.at[idx3_v.at[2]], rows_v.at[2], sem).wait()
        pltpu.sync_copy(rows_v, out_hbm.at[:, pl.ds(base, QPW)])

    return k(yst_flat, mkey)


def kernel(X, xs, ys):
    m = _tc_match_argmin(X.T, xs.T)
    out3q = _sc_gather(ys.T.reshape(-1), m)
    return out3q.T


# speedup vs baseline: 1.6088x; 1.0429x over previous
import functools

import jax
import jax.numpy as jnp
from jax import lax
from jax.experimental import pallas as pl
from jax.experimental.pallas import tpu as pltpu
from jax.experimental.pallas import tpu_sc as plsc

N, D, Q = 16384, 64, 512
NBLK = 2048
INV = 1.0 / 16384.0

SC_NC, SC_NS = 2, 16
SC_NW = SC_NC * SC_NS
QPW = Q // SC_NW
L = 16


def _match_argmin_body(xt_ref, xst_ref, out_ref, acc_ref):
    blk = pl.program_id(0)
    xqt = xt_ref[...]
    xbt = xst_ref[...]
    qb2t = (xqt + xqt).astype(jnp.bfloat16)
    dbt = xbt.astype(jnp.bfloat16)
    g2 = lax.dot_general(qb2t, dbt, (((0,), (0,)), ((), ())),
                         preferred_element_type=jnp.float32)
    sqq = xqt * xqt
    ones_col = jnp.ones((D, 1), jnp.float32)
    qn = lax.dot_general(sqq, ones_col, (((0,), (0,)), ((), ())),
                         preferred_element_type=jnp.float32)
    sq = dbt * dbt
    ones_row = jnp.ones((1, D), jnp.bfloat16)
    xn = lax.dot_general(ones_row, sq, (((1,), (0,)), ((), ())),
                         preferred_element_type=jnp.float32)
    iotaf = lax.broadcasted_iota(jnp.int32, (1, NBLK), 1).astype(
        jnp.float32) * INV + blk.astype(jnp.float32) * (NBLK * INV)
    key = ((qn + xn) - g2) + iotaf
    m = jnp.min(key, axis=1, keepdims=True)

    @pl.when(blk == 0)
    def _():
        acc_ref[...] = m

    @pl.when(blk > 0)
    def _():
        acc_ref[...] = jnp.minimum(acc_ref[...], m)

    @pl.when(blk == N // NBLK - 1)
    def _():
        out_ref[...] = acc_ref[...][:, 0]


def _tc_match_argmin(Xt, xst, interpret=False):
    grid = (N // NBLK,)
    return pl.pallas_call(
        _match_argmin_body,
        grid=grid,
        in_specs=[
            pl.BlockSpec((D, Q), lambda i: (0, 0)),
            pl.BlockSpec((D, NBLK), lambda i: (0, i)),
        ],
        out_specs=pl.BlockSpec((Q,), lambda i: (0,)),
        out_shape=jax.ShapeDtypeStruct((Q,), jnp.float32),
        scratch_shapes=[pltpu.VMEM((Q, 1), jnp.float32)],
        interpret=interpret,
    )(Xt, xst)


def _sc_gather(yst_flat, mkey):
    mesh = plsc.VectorSubcoreMesh(core_axis_name="c", subcore_axis_name="s")

    @functools.partial(
        pl.kernel,
        mesh=mesh,
        compiler_params=pltpu.CompilerParams(use_tc_tiling_on_sc=False),
        out_type=jax.ShapeDtypeStruct((3, Q), jnp.float32),
        scratch_types=[
            pltpu.VMEM((QPW,), jnp.float32),
            pltpu.VMEM((3, QPW), jnp.int32),
            pltpu.VMEM((3, QPW), jnp.float32),
            pltpu.SemaphoreType.DMA,
        ],
    )
    def k(ys_hbm, mkey_hbm, out_hbm, mk_v, idx3_v, rows_v, sem):
        wid = lax.axis_index("s") * SC_NC + lax.axis_index("c")
        base = wid * QPW
        pltpu.sync_copy(mkey_hbm.at[pl.ds(base, QPW)], mk_v)
        v = jnp.clip((mk_v[...] * 16384.0).astype(jnp.int32), 0, N - 1)
        for c in range(3):
            idx3_v[c, :] = v + c * N
            pltpu.async_copy(ys_hbm.at[idx3_v.at[c]], rows_v.at[c], sem)
        pltpu.make_async_copy(ys_hbm.at[idx3_v.at[0]], rows_v.at[0], sem).wait()
        pltpu.make_async_copy(ys_hbm.at[idx3_v.at[1]], rows_v.at[1], sem).wait()
        pltpu.make_async_copy(ys_hbm.at[idx3_v.at[2]], rows_v.at[2], sem).wait()
        pltpu.sync_copy(rows_v, out_hbm.at[:, pl.ds(base, QPW)])

    return k(yst_flat, mkey)


def kernel(X, xs, ys):
    m = _tc_match_argmin(X.T, xs.T)
    out3q = _sc_gather(ys.T.reshape(-1), m)
    return out3q.T


# speedup vs baseline: 1.6793x; 1.0438x over previous
import functools

import jax
import jax.numpy as jnp
from jax import lax
from jax.experimental import pallas as pl
from jax.experimental.pallas import tpu as pltpu
from jax.experimental.pallas import tpu_sc as plsc

N, D, Q = 16384, 64, 512
NBLK = 4096
INV = 1.0 / 16384.0

SC_NC, SC_NS = 2, 16
SC_NW = SC_NC * SC_NS
QPW = Q // SC_NW
L = 16


def _match_argmin_body(xt_ref, xst_ref, out_ref, acc_ref):
    blk = pl.program_id(0)
    xqt = xt_ref[...]
    xbt = xst_ref[...]
    qb2t = (xqt + xqt).astype(jnp.bfloat16)
    dbt = xbt.astype(jnp.bfloat16)
    g2 = lax.dot_general(qb2t, dbt, (((0,), (0,)), ((), ())),
                         preferred_element_type=jnp.float32)
    sqq = xqt * xqt
    ones_col = jnp.ones((D, 1), jnp.float32)
    qn = lax.dot_general(sqq, ones_col, (((0,), (0,)), ((), ())),
                         preferred_element_type=jnp.float32)
    sq = dbt * dbt
    ones_row = jnp.ones((1, D), jnp.bfloat16)
    xn = lax.dot_general(ones_row, sq, (((1,), (0,)), ((), ())),
                         preferred_element_type=jnp.float32)
    iotaf = lax.broadcasted_iota(jnp.int32, (1, NBLK), 1).astype(
        jnp.float32) * INV + blk.astype(jnp.float32) * (NBLK * INV)
    key = ((qn + xn) - g2) + iotaf
    m = jnp.min(key, axis=1, keepdims=True)

    @pl.when(blk == 0)
    def _():
        acc_ref[...] = m

    @pl.when(blk > 0)
    def _():
        acc_ref[...] = jnp.minimum(acc_ref[...], m)

    @pl.when(blk == N // NBLK - 1)
    def _():
        out_ref[...] = acc_ref[...][:, 0]


def _tc_match_argmin(Xt, xst, interpret=False):
    grid = (N // NBLK,)
    return pl.pallas_call(
        _match_argmin_body,
        grid=grid,
        in_specs=[
            pl.BlockSpec((D, Q), lambda i: (0, 0)),
            pl.BlockSpec((D, NBLK), lambda i: (0, i)),
        ],
        out_specs=pl.BlockSpec((Q,), lambda i: (0,)),
        out_shape=jax.ShapeDtypeStruct((Q,), jnp.float32),
        scratch_shapes=[pltpu.VMEM((Q, 1), jnp.float32)],
        interpret=interpret,
    )(Xt, xst)


def _sc_gather(yst_flat, mkey):
    mesh = plsc.VectorSubcoreMesh(core_axis_name="c", subcore_axis_name="s")

    @functools.partial(
        pl.kernel,
        mesh=mesh,
        compiler_params=pltpu.CompilerParams(use_tc_tiling_on_sc=False),
        out_type=jax.ShapeDtypeStruct((3, Q), jnp.float32),
        scratch_types=[
            pltpu.VMEM((QPW,), jnp.float32),
            pltpu.VMEM((3, QPW), jnp.int32),
            pltpu.VMEM((3, QPW), jnp.float32),
            pltpu.SemaphoreType.DMA,
        ],
    )
    def k(ys_hbm, mkey_hbm, out_hbm, mk_v, idx3_v, rows_v, sem):
        wid = lax.axis_index("s") * SC_NC + lax.axis_index("c")
        base = wid * QPW
        pltpu.sync_copy(mkey_hbm.at[pl.ds(base, QPW)], mk_v)
        v = jnp.clip((mk_v[...] * 16384.0).astype(jnp.int32), 0, N - 1)
        for c in range(3):
            idx3_v[c, :] = v + c * N
            pltpu.async_copy(ys_hbm.at[idx3_v.at[c]], rows_v.at[c], sem)
        pltpu.make_async_copy(ys_hbm.at[idx3_v.at[0]], rows_v.at[0], sem).wait()
        pltpu.make_async_copy(ys_hbm.at[idx3_v.at[1]], rows_v.at[1], sem).wait()
        pltpu.make_async_copy(ys_hbm.at[idx3_v.at[2]], rows_v.at[2], sem).wait()
        pltpu.sync_copy(rows_v, out_hbm.at[:, pl.ds(base, QPW)])

    return k(yst_flat, mkey)


def kernel(X, xs, ys):
    m = _tc_match_argmin(X.T, xs.T)
    out3q = _sc_gather(ys.T.reshape(-1), m)
    return out3q.T
